# baseline (speedup 1.0000x reference)
import dataclasses
import functools

import jax
import jax.numpy as jnp
from jax import lax
from jax.experimental import pallas as pl
from jax.experimental.pallas import tpu as pltpu
from jax.experimental.pallas import tpu_sc as plsc

B, H, W, C = 4, 512, 512, 21
N = H * W
K = N // 4
RB = 32

NT = 16
CHUNK = N // NT
LPT = CHUNK // 16
RPC = B // 2
KT = K // NT
NB = 256


def _unc_body(x_ref, o_ref):
    xs = [x_ref[0, c] for c in range(C)]
    m1 = jnp.maximum(xs[0], xs[1])
    m2 = jnp.minimum(xs[0], xs[1])
    for c in range(2, C):
        m2 = jnp.maximum(m2, jnp.minimum(m1, xs[c]))
        m1 = jnp.maximum(m1, xs[c])
    es = [jnp.exp(xs[c] - m1) for c in range(C)]
    w = []
    for s in range(8):
        t = es[s] + es[s + 8]
        if s + 16 < C:
            t = t + es[s + 16]
        w.append(t)
    z = ((w[0] + w[4]) + (w[2] + w[6])) + ((w[1] + w[5]) + (w[3] + w[7]))
    p2 = jnp.exp(m2 - m1) / z
    p1 = jnp.float32(1.0) / z
    u = p2 - p1
    b = lax.bitcast_convert_type(u, jnp.int32)
    o_ref[0] = jnp.where(b < 0, b, (~b) & jnp.int32(0x7FFFFFFF))


def _keys(inputs):
    xt = jnp.transpose(inputs, (0, 3, 1, 2))
    return pl.pallas_call(
        _unc_body,
        grid=(B, H // RB),
        in_specs=[pl.BlockSpec((1, C, RB, W), lambda b, i: (b, 0, i, 0))],
        out_specs=pl.BlockSpec((1, RB, W), lambda b, i: (b, i, 0)),
        out_shape=jax.ShapeDtypeStruct((B, H, W), jnp.int32),
    )(xt)


def _sort_body(keys_hbm, idx_out, cx_out, cy_out,
               bufv0, bufv1, grid_sp,
               keys_t, vals_t, hist, offs, tot_t, tot2,
               col_below, digit_base, vq, dq, cxs, cys, grid_v, sem):
    cid = lax.axis_index("c")
    tid = lax.axis_index("s")
    lane = lax.iota(jnp.int32, 16)
    zeros16 = jnp.zeros((16,), jnp.int32)
    ones16 = jnp.ones((16,), jnp.int32)

    for row in range(RPC):
        r = cid * RPC + row
        for p in range(4):
            shift = 8 * p
            srcv = (bufv1, bufv0, bufv1, bufv0)[p]
            dstv = (bufv0, bufv1, bufv0, bufv1)[p]

            @pl.loop(0, NB)
            def _(d):
                hist[pl.ds(d * 16, 16)] = zeros16

            if p == 0:
                pltpu.sync_copy(
                    keys_hbm.at[pl.ds(r * N + tid * CHUNK, CHUNK)], keys_t)
            else:
                pltpu.sync_copy(srcv.at[pl.ds(tid * CHUNK, CHUNK)], vals_t)
                pltpu.async_copy(keys_hbm.at[vals_t], keys_t, sem).wait()

            @pl.loop(0, LPT, step=4)
            def _(v0):
                for uu in range(4):
                    v = v0 + uu
                    idx = lane * LPT + v
                    k = plsc.load_gather(keys_t, [idx])
                    d = (k >> shift) & 0xFF
                    plsc.addupdate_scatter(hist, [d * 16 + lane], ones16)

            @pl.loop(0, NB)
            def _(d):
                hvec = hist[pl.ds(d * 16, 16)]
                offs[pl.ds(d * 16, 16)] = plsc.cumsum(hvec) - hvec

            @pl.loop(0, 16)
            def _(dc):
                base = (dc * 16 + lane) * 16
                acc = zeros16
                for l in range(16):
                    acc = acc + plsc.load_gather(hist, [base + l])
                tot_t[pl.ds(dc * 16, 16)] = acc

            pltpu.sync_copy(tot_t, grid_sp.at[tid])
            plsc.subcore_barrier()

            pltpu.sync_copy(grid_sp, grid_v)
            for dc in range(16):
                accb = zeros16
                acct = zeros16
                for t in range(16):
                    rowv = grid_v[t, pl.ds(dc * 16, 16)]
                    acct = acct + rowv
                    m = jnp.where(t < tid, 1, 0)
                    accb = accb + rowv * m
                tot2[pl.ds(dc * 16, 16)] = acct
                col_below[pl.ds(dc * 16, 16)] = accb
            carry = jnp.int32(0)
            for dc in range(16):
                ch = tot2[pl.ds(dc * 16, 16)]
                cs = plsc.cumsum(ch)
                digit_base[pl.ds(dc * 16, 16)] = (cs - ch + carry
                                                  + col_below[pl.ds(dc * 16, 16)])
                carry = carry + jnp.sum(ch)

            @pl.loop(0, NB)
            def _(d):
                s = plsc.load_gather(digit_base, [zeros16 + d])
                offs[pl.ds(d * 16, 16)] = offs[pl.ds(d * 16, 16)] + s

            @pl.loop(0, LPT, step=4)
            def _(v0):
                for uu in range(4):
                    v = v0 + uu
                    idx = lane * LPT + v
                    k = plsc.load_gather(keys_t, [idx])
                    if p == 0:
                        val = r * N + tid * CHUNK + idx
                    else:
                        val = plsc.load_gather(vals_t, [idx])
                    d = (k >> shift) & 0xFF
                    bin_ = d * 16 + lane
                    pos = plsc.load_gather(offs, [bin_])
                    plsc.store_scatter(offs, [bin_], pos + 1)
                    dq[pl.ds(v * 16, 16)] = pos
                    vq[pl.ds(v * 16, 16)] = val

            pltpu.async_copy(vq, dstv.at[dq], sem).wait()

            plsc.subcore_barrier()

        pltpu.sync_copy(bufv1.at[pl.ds(tid * KT, KT)],
                        vals_t.at[pl.ds(0, KT)])

        @pl.loop(0, KT // 16)
        def _(i):
            v = vals_t[pl.ds(i * 16, 16)] - r * N
            vq[pl.ds(i * 16, 16)] = v
            xf = (v & (W - 1)).astype(jnp.float32)
            yf = (v >> 9).astype(jnp.float32)
            cxs[pl.ds(i * 16, 16)] = (jnp.float32(0.5 / W)
                                      + xf * jnp.float32(1.0 / W))
            cys[pl.ds(i * 16, 16)] = (jnp.float32(0.5 / H)
                                      + yf * jnp.float32(1.0 / H))

        pltpu.sync_copy(vq.at[pl.ds(0, KT)],
                        idx_out.at[r, pl.ds(tid * KT, KT)])
        pltpu.sync_copy(cxs, cx_out.at[r, pl.ds(tid * KT, KT)])
        pltpu.sync_copy(cys, cy_out.at[r, pl.ds(tid * KT, KT)])
        plsc.subcore_barrier()


def _topk_sc(keys):
    mesh = plsc.VectorSubcoreMesh(core_axis_name="c", subcore_axis_name="s")
    cp = pltpu.CompilerParams()
    if "needs_layout_passes" in pltpu.CompilerParams.__dataclass_fields__:
        cp = dataclasses.replace(cp, needs_layout_passes=False)
    kern = functools.partial(
        pl.kernel,
        compiler_params=cp,
        out_type=[
            jax.ShapeDtypeStruct((B, K), jnp.int32),
            jax.ShapeDtypeStruct((B, K), jnp.float32),
            jax.ShapeDtypeStruct((B, K), jnp.float32),
        ],
        mesh=mesh,
        scratch_types=[
            pltpu.VMEM_SHARED((N,), jnp.int32),
            pltpu.VMEM_SHARED((N,), jnp.int32),
            pltpu.VMEM_SHARED((NT, NB), jnp.int32),
            pltpu.VMEM((CHUNK,), jnp.int32),
            pltpu.VMEM((CHUNK,), jnp.int32),
            pltpu.VMEM((NB * 16,), jnp.int32),
            pltpu.VMEM((NB * 16,), jnp.int32),
            pltpu.VMEM((NB,), jnp.int32),
            pltpu.VMEM((NB,), jnp.int32),
            pltpu.VMEM((NB,), jnp.int32),
            pltpu.VMEM((NB,), jnp.int32),
            pltpu.VMEM((CHUNK,), jnp.int32),
            pltpu.VMEM((CHUNK,), jnp.int32),
            pltpu.VMEM((KT,), jnp.float32),
            pltpu.VMEM((KT,), jnp.float32),
            pltpu.VMEM((NT, NB), jnp.int32),
            pltpu.SemaphoreType.DMA,
        ],
    )(_sort_body)
    return kern(keys)


def kernel(inputs):
    keys = _keys(inputs).reshape(B * N)
    idx, cx, cy = _topk_sc(keys)
    point_coords = jnp.stack([cx, cy], axis=-1)
    return (idx, point_coords)


# speedup vs baseline: 1.0898x; 1.0898x over previous
import dataclasses
import functools

import jax
import jax.numpy as jnp
from jax import lax
from jax.experimental import pallas as pl
from jax.experimental.pallas import tpu as pltpu
from jax.experimental.pallas import tpu_sc as plsc

B, H, W, C = 4, 512, 512, 21
N = H * W
K = N // 4
RB = 32

NT = 16
CHUNK = N // NT
LPT = CHUNK // 16
RPC = B // 2
KT = K // NT
NB = 256


def _unc_body(x_ref, o_ref):
    xs = [x_ref[0, c] for c in range(C)]
    m1 = jnp.maximum(xs[0], xs[1])
    m2 = jnp.minimum(xs[0], xs[1])
    for c in range(2, C):
        m2 = jnp.maximum(m2, jnp.minimum(m1, xs[c]))
        m1 = jnp.maximum(m1, xs[c])
    es = [jnp.exp(xs[c] - m1) for c in range(C)]
    w = []
    for s in range(8):
        t = es[s] + es[s + 8]
        if s + 16 < C:
            t = t + es[s + 16]
        w.append(t)
    z = ((w[0] + w[4]) + (w[2] + w[6])) + ((w[1] + w[5]) + (w[3] + w[7]))
    p2 = jnp.exp(m2 - m1) / z
    p1 = jnp.float32(1.0) / z
    u = p2 - p1
    b = lax.bitcast_convert_type(u, jnp.int32)
    o_ref[0] = jnp.where(b < 0, b, (~b) & jnp.int32(0x7FFFFFFF))


def _keys(inputs):
    xt = jnp.transpose(inputs, (0, 3, 1, 2))
    return pl.pallas_call(
        _unc_body,
        grid=(B, H // RB),
        in_specs=[pl.BlockSpec((1, C, RB, W), lambda b, i: (b, 0, i, 0))],
        out_specs=pl.BlockSpec((1, RB, W), lambda b, i: (b, i, 0)),
        out_shape=jax.ShapeDtypeStruct((B, H, W), jnp.int32),
    )(xt)


def _sort_body(keys_hbm, idx_out, cx_out, cy_out,
               bufv0, bufv1, grid_sp,
               keys_t, vals_t, hist, offs, tot_t, tot2,
               col_below, digit_base, vq, dq, cxs, cys, grid_v, sem):
    cid = lax.axis_index("c")
    tid = lax.axis_index("s")
    lane = lax.iota(jnp.int32, 16)
    zeros16 = jnp.zeros((16,), jnp.int32)
    ones16 = jnp.ones((16,), jnp.int32)

    for row in range(RPC):
        r = cid * RPC + row
        for p in range(4):
            shift = 8 * p
            srcv = (bufv1, bufv0, bufv1, bufv0)[p]
            dstv = (bufv0, bufv1, bufv0, bufv1)[p]

            @pl.loop(0, NB)
            def _(d):
                hist[pl.ds(d * 16, 16)] = zeros16

            if p == 0:
                pltpu.sync_copy(
                    keys_hbm.at[pl.ds(r * N + tid * CHUNK, CHUNK)], keys_t)
            else:
                pltpu.sync_copy(srcv.at[pl.ds(tid * CHUNK, CHUNK)], vals_t)
                pltpu.async_copy(keys_hbm.at[vals_t], keys_t, sem).wait()

            @pl.loop(0, LPT)
            def _(v):
                idx = lane * LPT + v
                k = plsc.load_gather(keys_t, [idx])
                d = (k >> shift) & 0xFF
                plsc.addupdate_scatter(hist, [d * 16 + lane], ones16)

            @pl.loop(0, NB)
            def _(d):
                hvec = hist[pl.ds(d * 16, 16)]
                offs[pl.ds(d * 16, 16)] = plsc.cumsum(hvec) - hvec

            @pl.loop(0, 16)
            def _(dc):
                base = (dc * 16 + lane) * 16
                acc = zeros16
                for l in range(16):
                    acc = acc + plsc.load_gather(hist, [base + l])
                tot_t[pl.ds(dc * 16, 16)] = acc

            pltpu.sync_copy(tot_t, grid_sp.at[tid])
            plsc.subcore_barrier()

            pltpu.sync_copy(grid_sp, grid_v)
            for dc in range(16):
                accb = zeros16
                acct = zeros16
                for t in range(16):
                    rowv = grid_v[t, pl.ds(dc * 16, 16)]
                    acct = acct + rowv
                    m = jnp.where(t < tid, 1, 0)
                    accb = accb + rowv * m
                tot2[pl.ds(dc * 16, 16)] = acct
                col_below[pl.ds(dc * 16, 16)] = accb
            carry = jnp.int32(0)
            for dc in range(16):
                ch = tot2[pl.ds(dc * 16, 16)]
                cs = plsc.cumsum(ch)
                digit_base[pl.ds(dc * 16, 16)] = (cs - ch + carry
                                                  + col_below[pl.ds(dc * 16, 16)])
                carry = carry + jnp.sum(ch)

            @pl.loop(0, NB)
            def _(d):
                s = plsc.load_gather(digit_base, [zeros16 + d])
                offs[pl.ds(d * 16, 16)] = offs[pl.ds(d * 16, 16)] + s

            @pl.loop(0, LPT)
            def _(v):
                idx = lane * LPT + v
                k = plsc.load_gather(keys_t, [idx])
                if p == 0:
                    val = r * N + tid * CHUNK + idx
                else:
                    val = plsc.load_gather(vals_t, [idx])
                d = (k >> shift) & 0xFF
                bin_ = d * 16 + lane
                pos = plsc.load_gather(offs, [bin_])
                plsc.store_scatter(offs, [bin_], pos + 1)
                dq[pl.ds(v * 16, 16)] = pos
                vq[pl.ds(v * 16, 16)] = val

            pltpu.async_copy(vq, dstv.at[dq], sem).wait()

            plsc.subcore_barrier()

        pltpu.sync_copy(bufv1.at[pl.ds(tid * KT, KT)],
                        vals_t.at[pl.ds(0, KT)])

        @pl.loop(0, KT // 16)
        def _(i):
            v = vals_t[pl.ds(i * 16, 16)] - r * N
            vq[pl.ds(i * 16, 16)] = v
            xf = (v & (W - 1)).astype(jnp.float32)
            yf = (v >> 9).astype(jnp.float32)
            cxs[pl.ds(i * 16, 16)] = (jnp.float32(0.5 / W)
                                      + xf * jnp.float32(1.0 / W))
            cys[pl.ds(i * 16, 16)] = (jnp.float32(0.5 / H)
                                      + yf * jnp.float32(1.0 / H))

        pltpu.sync_copy(vq.at[pl.ds(0, KT)],
                        idx_out.at[r, pl.ds(tid * KT, KT)])
        pltpu.sync_copy(cxs, cx_out.at[r, pl.ds(tid * KT, KT)])
        pltpu.sync_copy(cys, cy_out.at[r, pl.ds(tid * KT, KT)])
        plsc.subcore_barrier()


def _topk_sc(keys):
    mesh = plsc.VectorSubcoreMesh(core_axis_name="c", subcore_axis_name="s")
    cp = pltpu.CompilerParams()
    if "needs_layout_passes" in pltpu.CompilerParams.__dataclass_fields__:
        cp = dataclasses.replace(cp, needs_layout_passes=False)
    kern = functools.partial(
        pl.kernel,
        compiler_params=cp,
        out_type=[
            jax.ShapeDtypeStruct((B, K), jnp.int32),
            jax.ShapeDtypeStruct((B, K), jnp.float32),
            jax.ShapeDtypeStruct((B, K), jnp.float32),
        ],
        mesh=mesh,
        scratch_types=[
            pltpu.VMEM_SHARED((N,), jnp.int32),
            pltpu.VMEM_SHARED((N,), jnp.int32),
            pltpu.VMEM_SHARED((NT, NB), jnp.int32),
            pltpu.VMEM((CHUNK,), jnp.int32),
            pltpu.VMEM((CHUNK,), jnp.int32),
            pltpu.VMEM((NB * 16,), jnp.int32),
            pltpu.VMEM((NB * 16,), jnp.int32),
            pltpu.VMEM((NB,), jnp.int32),
            pltpu.VMEM((NB,), jnp.int32),
            pltpu.VMEM((NB,), jnp.int32),
            pltpu.VMEM((NB,), jnp.int32),
            pltpu.VMEM((CHUNK,), jnp.int32),
            pltpu.VMEM((CHUNK,), jnp.int32),
            pltpu.VMEM((KT,), jnp.float32),
            pltpu.VMEM((KT,), jnp.float32),
            pltpu.VMEM((NT, NB), jnp.int32),
            pltpu.SemaphoreType.DMA,
        ],
    )(_sort_body)
    return kern(keys)


def kernel(inputs):
    keys = _keys(inputs).reshape(B * N)
    idx, cx, cy = _topk_sc(keys)
    point_coords = jnp.stack([cx, cy], axis=-1)
    return (idx, point_coords)
